# baseline (speedup 1.0000x reference)
import functools
import math

import jax
import jax.numpy as jnp
from jax import lax
from jax.experimental import pallas as pl
from jax.experimental.pallas import tpu as pltpu
from jax.experimental.pallas import tpu_sc as plsc

D_MODEL = 64
SCALE = math.sqrt(D_MODEL)

NUM_CORES = 2
NUM_SUBCORES = 16
NUM_WORKERS = NUM_CORES * NUM_SUBCORES

CHUNK = 512


def _emb_body(x_hbm, lut_hbm, out_hbm, idx_v, rows_v, gsem):
    wid = lax.axis_index("s") * NUM_CORES + lax.axis_index("c")
    n_total = x_hbm.shape[0]
    b_per_w = n_total // NUM_WORKERS
    n_chunks = b_per_w // CHUNK
    base = wid * b_per_w

    def chunk_body(g, carry):
        off = base + g * CHUNK
        pltpu.sync_copy(x_hbm.at[pl.ds(off, CHUNK)], idx_v)
        pltpu.async_copy(lut_hbm.at[idx_v], rows_v, gsem).wait()

        def row_body(r, c2):
            for k in range(D_MODEL // 16):
                sl = pl.ds(16 * k, 16)
                rows_v[r, sl] = rows_v[r, sl] * SCALE
            return c2

        lax.fori_loop(0, CHUNK, row_body, 0, unroll=2)
        pltpu.sync_copy(rows_v, out_hbm.at[pl.ds(off, CHUNK)])
        return carry

    lax.fori_loop(0, n_chunks, chunk_body, 0)


def kernel(x, lut):
    b, s = x.shape
    n = b * s
    xf = x.reshape(n).astype(jnp.int32)

    emb_call = pl.kernel(
        _emb_body,
        out_type=jax.ShapeDtypeStruct((n, D_MODEL), jnp.float32),
        mesh=plsc.VectorSubcoreMesh(
            core_axis_name="c", subcore_axis_name="s",
            num_cores=NUM_CORES, num_subcores=NUM_SUBCORES,
        ),
        scratch_types=[
            pltpu.VMEM((CHUNK,), jnp.int32),
            pltpu.VMEM((CHUNK, D_MODEL), jnp.float32),
            pltpu.SemaphoreType.DMA,
        ],
        compiler_params=pltpu.CompilerParams(use_tc_tiling_on_sc=False),
    )
    out = emb_call(xf, lut)
    return out.reshape(b, s, D_MODEL)


# speedup vs baseline: 1.0938x; 1.0938x over previous
import functools
import math

import jax
import jax.numpy as jnp
from jax import lax
from jax.experimental import pallas as pl
from jax.experimental.pallas import tpu as pltpu
from jax.experimental.pallas import tpu_sc as plsc

D_MODEL = 64
SCALE = math.sqrt(D_MODEL)

NUM_CORES = 2
NUM_SUBCORES = 16
NUM_WORKERS = NUM_CORES * NUM_SUBCORES

CHUNK = 400


def _emb_body(x_hbm, lut_hbm, out_hbm, idx_all, g0, g1, o0, o1, gsem, osem):
    wid = lax.axis_index("s") * NUM_CORES + lax.axis_index("c")
    n_total = x_hbm.shape[0]
    b_per_w = n_total // NUM_WORKERS
    n_chunks = b_per_w // CHUNK
    base = wid * b_per_w

    grows = [g0, g1]
    orows = [o0, o1]

    pltpu.sync_copy(x_hbm.at[pl.ds(base, b_per_w)], idx_all)
    pltpu.async_copy(lut_hbm.at[idx_all.at[pl.ds(0, CHUNK)]], grows[0], gsem)
    pltpu.async_copy(lut_hbm.at[idx_all.at[pl.ds(CHUNK, CHUNK)]], grows[1],
                     gsem)

    def half_step(g, s):
        gsl = idx_all.at[pl.ds(g * CHUNK, CHUNK)]
        pltpu.make_async_copy(lut_hbm.at[gsl], grows[s], gsem).wait()

        @pl.when(g >= 2)
        def _():
            pltpu.make_async_copy(
                orows[s], out_hbm.at[pl.ds(base, CHUNK)], osem).wait()

        @plsc.parallel_loop(0, CHUNK, unroll=4)
        def _(r):
            for k in range(D_MODEL // 16):
                sl = pl.ds(16 * k, 16)
                orows[s][r, sl] = grows[s][r, sl] * SCALE

        @pl.when(g + 2 < n_chunks)
        def _():
            nsl = idx_all.at[pl.ds((g + 2) * CHUNK, CHUNK)]
            pltpu.async_copy(lut_hbm.at[nsl], grows[s], gsem)

        pltpu.async_copy(orows[s], out_hbm.at[pl.ds(base + g * CHUNK, CHUNK)],
                         osem)

    def pair_body(i, carry):
        half_step(2 * i, 0)
        half_step(2 * i + 1, 1)
        return carry

    lax.fori_loop(0, n_chunks // 2, pair_body, 0)

    for s in range(2):
        pltpu.make_async_copy(
            orows[s], out_hbm.at[pl.ds(base, CHUNK)], osem).wait()


def kernel(x, lut):
    b, s = x.shape
    n = b * s
    xf = x.reshape(n).astype(jnp.int32)

    emb_call = pl.kernel(
        _emb_body,
        out_type=jax.ShapeDtypeStruct((n, D_MODEL), jnp.float32),
        mesh=plsc.VectorSubcoreMesh(
            core_axis_name="c", subcore_axis_name="s",
            num_cores=NUM_CORES, num_subcores=NUM_SUBCORES,
        ),
        scratch_types=[
            pltpu.VMEM((n // NUM_WORKERS,), jnp.int32),
            pltpu.VMEM((CHUNK, D_MODEL), jnp.float32),
            pltpu.VMEM((CHUNK, D_MODEL), jnp.float32),
            pltpu.VMEM((CHUNK, D_MODEL), jnp.float32),
            pltpu.VMEM((CHUNK, D_MODEL), jnp.float32),
            pltpu.SemaphoreType.DMA,
            pltpu.SemaphoreType.DMA,
        ],
        compiler_params=pltpu.CompilerParams(use_tc_tiling_on_sc=False),
    )
    out = emb_call(xf, lut)
    return out.reshape(b, s, D_MODEL)


# speedup vs baseline: 1.2520x; 1.1447x over previous
import functools
import math

import jax
import jax.numpy as jnp
from jax import lax
from jax.experimental import pallas as pl
from jax.experimental.pallas import tpu as pltpu
from jax.experimental.pallas import tpu_sc as plsc

D_MODEL = 64
SCALE = math.sqrt(D_MODEL)

NUM_CORES = 2
NUM_SUBCORES = 16
NUM_WORKERS = NUM_CORES * NUM_SUBCORES

CHUNK = 160


def _emb_body(x_hbm, lut_hbm, out_hbm, idx_all, p0, p1, g0, g1, o0, o1,
              gsem, osem):
    wid = lax.axis_index("s") * NUM_CORES + lax.axis_index("c")
    n_total = x_hbm.shape[0]
    b_per_w = n_total // NUM_WORKERS
    n_chunks = b_per_w // CHUNK
    base = wid * b_per_w

    pidx = [p0, p1]
    grows = [g0, g1]
    orows = [o0, o1]

    pltpu.sync_copy(x_hbm.at[pl.ds(base, b_per_w)],
                    idx_all.at[pl.ds(0, b_per_w)])

    def fill_pidx(g, s):
        def vec(i, c):
            v = idx_all[pl.ds(g * CHUNK + i * 16, 16)]
            pidx[s][pl.ds(i * 16, 16)] = jax.lax.shift_right_logical(v, 1)
            return c
        lax.fori_loop(0, CHUNK // 16, vec, 0, unroll=4)

    fill_pidx(0, 0)
    pltpu.async_copy(lut_hbm.at[pidx[0]], grows[0], gsem)
    fill_pidx(1, 1)
    pltpu.async_copy(lut_hbm.at[pidx[1]], grows[1], gsem)

    def half_step(g, s):
        pltpu.make_async_copy(lut_hbm.at[pidx[s]], grows[s], gsem).wait()

        @pl.when(g >= 2)
        def _():
            pltpu.make_async_copy(
                orows[s], out_hbm.at[pl.ds(base, CHUNK)], osem).wait()

        @plsc.parallel_loop(0, CHUNK, unroll=4)
        def _(r):
            v = idx_all[pl.ds(g * CHUNK + r, 16)]
            par = v[0] & 1
            off = par * D_MODEL
            for k in range(D_MODEL // 16):
                orows[s][r, pl.ds(16 * k, 16)] = (
                    grows[s][r, pl.ds(off + 16 * k, 16)] * SCALE)

        @pl.when(g + 2 < n_chunks)
        def _():
            fill_pidx(g + 2, s)
            pltpu.async_copy(lut_hbm.at[pidx[s]], grows[s], gsem)

        pltpu.async_copy(orows[s], out_hbm.at[pl.ds(base + g * CHUNK, CHUNK)],
                         osem)

    def pair_body(i, carry):
        half_step(2 * i, 0)
        half_step(2 * i + 1, 1)
        return carry

    lax.fori_loop(0, n_chunks // 2, pair_body, 0)

    for s in range(2):
        pltpu.make_async_copy(
            orows[s], out_hbm.at[pl.ds(base, CHUNK)], osem).wait()


def kernel(x, lut):
    b, s = x.shape
    n = b * s
    xf = x.reshape(n).astype(jnp.int32)
    lut2 = lut.reshape(lut.shape[0] // 2, 2 * lut.shape[1])

    emb_call = pl.kernel(
        _emb_body,
        out_type=jax.ShapeDtypeStruct((n, D_MODEL), jnp.float32),
        mesh=plsc.VectorSubcoreMesh(
            core_axis_name="c", subcore_axis_name="s",
            num_cores=NUM_CORES, num_subcores=NUM_SUBCORES,
        ),
        scratch_types=[
            pltpu.VMEM((n // NUM_WORKERS + 16,), jnp.int32),
            pltpu.VMEM((CHUNK,), jnp.int32),
            pltpu.VMEM((CHUNK,), jnp.int32),
            pltpu.VMEM((CHUNK, 2 * D_MODEL), jnp.float32),
            pltpu.VMEM((CHUNK, 2 * D_MODEL), jnp.float32),
            pltpu.VMEM((CHUNK, D_MODEL), jnp.float32),
            pltpu.VMEM((CHUNK, D_MODEL), jnp.float32),
            pltpu.SemaphoreType.DMA,
            pltpu.SemaphoreType.DMA,
        ],
        compiler_params=pltpu.CompilerParams(use_tc_tiling_on_sc=True),
    )
    out = emb_call(xf, lut2)
    return out.reshape(b, s, D_MODEL)
